# Initial kernel scaffold; baseline (speedup 1.0000x reference)
import jax
import jax.numpy as jnp
from jax.experimental import pallas as pl


def kernel(x):
    raise NotImplementedError("write your pallas kernel here")


# speedup vs baseline: 5.0720x; 5.0720x over previous
import functools

import jax
import jax.numpy as jnp
from jax.experimental import pallas as pl


def _cumsum_body(x_ref, o_ref, *, cb: int):
    rb, cols = x_ref.shape
    nchunk = cols // cb
    row = jax.lax.broadcasted_iota(jnp.int32, (cb, cb), 0)
    col = jax.lax.broadcasted_iota(jnp.int32, (cb, cb), 1)
    tri = (row <= col).astype(jnp.float32)

    def step(c, carry):
        blk = x_ref[:, pl.ds(c * cb, cb)]
        cs = jax.lax.dot(blk, tri, preferred_element_type=jnp.float32)
        out = cs + carry
        o_ref[:, pl.ds(c * cb, cb)] = out
        return out[:, cb - 1 : cb]

    jax.lax.fori_loop(0, nchunk, step, jnp.zeros((rb, 1), jnp.float32))


@jax.jit
def kernel(x):
    rows, cols = x.shape
    rb = 256
    cb = 256
    body = functools.partial(_cumsum_body, cb=cb)
    return pl.pallas_call(
        body,
        grid=(rows // rb,),
        in_specs=[pl.BlockSpec((rb, cols), lambda i: (i, 0))],
        out_specs=pl.BlockSpec((rb, cols), lambda i: (i, 0)),
        out_shape=jax.ShapeDtypeStruct((rows, cols), x.dtype),
    )(x)


# speedup vs baseline: 6.7393x; 1.3287x over previous
import functools

import jax
import jax.numpy as jnp
from jax.experimental import pallas as pl


def _cumsum_body(x_ref, o_ref, *, cb: int):
    rb, cols = x_ref.shape
    nchunk = cols // cb
    row = jax.lax.broadcasted_iota(jnp.int32, (cb, cb), 0)
    col = jax.lax.broadcasted_iota(jnp.int32, (cb, cb), 1)
    tri = (row <= col).astype(jnp.float32)

    carry = jnp.zeros((rb, 1), jnp.float32)
    for c in range(nchunk):
        blk = x_ref[:, c * cb : (c + 1) * cb]
        cs = jax.lax.dot(blk, tri, preferred_element_type=jnp.float32)
        o_ref[:, c * cb : (c + 1) * cb] = cs + carry
        carry = carry + cs[:, cb - 1 : cb]


@jax.jit
def kernel(x):
    rows, cols = x.shape
    rb = 256
    cb = 256
    body = functools.partial(_cumsum_body, cb=cb)
    return pl.pallas_call(
        body,
        grid=(rows // rb,),
        in_specs=[pl.BlockSpec((rb, cols), lambda i: (i, 0))],
        out_specs=pl.BlockSpec((rb, cols), lambda i: (i, 0)),
        out_shape=jax.ShapeDtypeStruct((rows, cols), x.dtype),
    )(x)
